# baseline (speedup 1.0000x reference)
import functools

import jax
import jax.numpy as jnp
from jax import lax
from jax.experimental import pallas as pl
from jax.experimental.pallas import tpu as pltpu
from jax.experimental.pallas import tpu_sc as plsc

D_MODEL = 64
N_IDX = 4096 * 200
LANE = 128
N_ROWS = N_IDX // LANE
NW = 32
ROWS_PER_W = N_ROWS // NW
H = 4
N_STEP = ROWS_PER_W // H


def _make_gather():
  mesh = plsc.VectorSubcoreMesh(
      core_axis_name="c", subcore_axis_name="s", num_cores=2, num_subcores=16
  )

  @functools.partial(
      pl.kernel,
      mesh=mesh,
      compiler_params=pltpu.CompilerParams(use_tc_tiling_on_sc=False),
      out_type=jax.ShapeDtypeStruct((N_IDX, D_MODEL), jnp.float32),
      scratch_types=[
          pltpu.VMEM((ROWS_PER_W, LANE), jnp.int32),
          pltpu.VMEM((H * LANE, D_MODEL), jnp.float32),
          pltpu.VMEM((H * LANE, D_MODEL), jnp.float32),
          pltpu.SemaphoreType.DMA,
          pltpu.SemaphoreType.DMA,
          pltpu.SemaphoreType.DMA,
          pltpu.SemaphoreType.DMA,
      ],
  )
  def gather_kernel(
      x_hbm, pe_hbm, out_hbm, idx_v, buf_a, buf_b, gsem_a, gsem_b, osem_a, osem_b
  ):
    wid = lax.axis_index("s") * 2 + lax.axis_index("c")
    row0 = wid * ROWS_PER_W

    pltpu.sync_copy(x_hbm.at[pl.ds(row0, ROWS_PER_W)], idx_v)

    def issue_gathers(s, buf, gsem):
      for k in range(H):
        pltpu.async_copy(
            pe_hbm.at[idx_v.at[s * H + k]], buf.at[pl.ds(k * LANE, LANE)], gsem
        )

    def wait_gathers(s, buf, gsem):
      for k in range(H):
        pltpu.make_async_copy(
            pe_hbm.at[idx_v.at[s * H + k]], buf.at[pl.ds(k * LANE, LANE)], gsem
        ).wait()

    def issue_out(s, buf, osem):
      pltpu.async_copy(
          buf, out_hbm.at[pl.ds((row0 + s * H) * LANE, H * LANE)], osem
      )

    def wait_out(s, buf, osem):
      pltpu.make_async_copy(
          buf, out_hbm.at[pl.ds((row0 + s * H) * LANE, H * LANE)], osem
      ).wait()

    issue_gathers(0, buf_a, gsem_a)

    def step(s, carry):
      def body(cur_buf, cur_g, cur_o, oth_buf, oth_g, oth_o):
        wait_gathers(s, cur_buf, cur_g)
        issue_out(s, cur_buf, cur_o)

        @pl.when(s < N_STEP - 1)
        def _():
          @pl.when(s > 0)
          def _():
            wait_out(s - 1, oth_buf, oth_o)

          issue_gathers(s + 1, oth_buf, oth_g)

      even = (s % 2) == 0

      @pl.when(even)
      def _():
        body(buf_a, gsem_a, osem_a, buf_b, gsem_b, osem_b)

      @pl.when(jnp.logical_not(even))
      def _():
        body(buf_b, gsem_b, osem_b, buf_a, gsem_a, osem_a)

      return carry

    lax.fori_loop(0, N_STEP, step, 0)

    wait_out(N_STEP - 2, buf_a, osem_a)
    wait_out(N_STEP - 1, buf_b, osem_b)

  return gather_kernel


def kernel(x, pe):
  xf = x.astype(jnp.int32).reshape(N_ROWS, LANE)
  out = _make_gather()(xf, pe)
  return out.reshape(4096, 200, D_MODEL)
